# baseline (speedup 1.0000x reference)
import functools

import jax
import jax.numpy as jnp
from jax import lax
from jax.experimental import pallas as pl
from jax.experimental.pallas import tpu as pltpu
from jax.experimental.pallas import tpu_sc as plsc

_N = 10000
_E = 320000
_NW = 32
_EPT = _E // _NW
_CH = 80
_NCH = _EPT // _CH


def _leaky(x, s):
    return jnp.where(x >= 0, x, s * x)


def _emit_nodes(hT, h_refs, s_ref, d_ref, c_ref, a):
    sd = lax.dot_general(a, hT, (((1,), (0,)), ((), ())),
                         preferred_element_type=jnp.float32)
    for i in range(4):
        h_refs[i][...] = hT[i]
    s_ref[...] = sd[0]
    d_ref[...] = sd[1]
    m = jnp.max(sd[0:1]) + jnp.max(sd[1:2])
    c_ref[...] = jnp.full((16,), _leaky(m, 0.2), jnp.float32)


def _tc_prep1_body(xT_ref, W1_ref, a_ref, h0_ref, h1_ref, h2_ref, h3_ref,
                   s_ref, d_ref, c_ref):
    hT = lax.dot_general(W1_ref[...], xT_ref[...], (((0,), (0,)), ((), ())),
                         preferred_element_type=jnp.float32)
    _emit_nodes(hT, (h0_ref, h1_ref, h2_ref, h3_ref), s_ref, d_ref, c_ref,
                a_ref[...])


def _tc_combine_body(part_ref, b_ref, W_ref, a_ref, h0_ref, h1_ref, h2_ref,
                     h3_ref, s_ref, d_ref, c_ref):
    acc = jnp.sum(part_ref[...], axis=0)
    outT = acc[0:4] / (acc[4:5] + 1e-16) + b_ref[...]
    hT = lax.dot_general(W_ref[...], outT, (((0,), (0,)), ((), ())),
                         preferred_element_type=jnp.float32)
    _emit_nodes(hT, (h0_ref, h1_ref, h2_ref, h3_ref), s_ref, d_ref, c_ref,
                a_ref[...])


def _tc_final_nodes_body(part_ref, b_ref, bw1_ref, bb1_ref, bw2_ref, bb2_ref,
                         bw3_ref, bb3_ref, bw4_ref, bb4_ref,
                         h0_ref, h1_ref, h2_ref, h3_ref, b0_ref):
    acc = jnp.sum(part_ref[...], axis=0)
    outT = acc[0:4] / (acc[4:5] + 1e-16) + b_ref[...]
    h0_ref[...] = outT[0]
    h1_ref[...] = outT[1]
    h2_ref[...] = outT[2]
    h3_ref[...] = outT[3]
    h0 = outT[:, 0:1]
    t = _leaky(lax.dot_general(bw1_ref[...], h0, (((0,), (0,)), ((), ())),
                               preferred_element_type=jnp.float32)
               + bb1_ref[...], 0.01)
    t = _leaky(lax.dot_general(bw2_ref[...], t, (((0,), (0,)), ((), ())),
                               preferred_element_type=jnp.float32)
               + bb2_ref[...], 0.01)
    t = _leaky(lax.dot_general(bw3_ref[...], t, (((0,), (0,)), ((), ())),
                               preferred_element_type=jnp.float32)
               + bb3_ref[...], 0.01)
    b0 = (lax.dot_general(bw4_ref[...], t, (((0,), (0,)), ((), ())),
                          preferred_element_type=jnp.float32)
          + bb4_ref[...])
    b0_ref[...] = b0


def _tc_edge_mlp_body(rc_ref, at_ref, ew1_ref, eb1_ref, ew2_ref, eb2_ref,
                      ew3_ref, eb3_ref, ew4_ref, eb4_ref, out_ref):
    ew1 = ew1_ref[...]
    t = lax.dot_general(ew1[0:8], rc_ref[...], (((0,), (0,)), ((), ())),
                        preferred_element_type=jnp.float32)
    t = t + lax.dot_general(ew1[8:9], at_ref[...], (((0,), (0,)), ((), ())),
                            preferred_element_type=jnp.float32)
    t = _leaky(t + eb1_ref[...], 0.01)
    t = _leaky(lax.dot_general(ew2_ref[...], t, (((0,), (0,)), ((), ())),
                               preferred_element_type=jnp.float32)
               + eb2_ref[...], 0.01)
    t = _leaky(lax.dot_general(ew3_ref[...], t, (((0,), (0,)), ((), ())),
                               preferred_element_type=jnp.float32)
               + eb3_ref[...], 0.01)
    out_ref[...] = (lax.dot_general(ew4_ref[...], t, (((0,), (0,)), ((), ())),
                                    preferred_element_type=jnp.float32)
                    + eb4_ref[...])


def _tc_ode_body(win_ref, wout_ref, b0_ref, x0_ref, out_ref):
    win = win_ref[...]
    wout = wout_ref[...]
    bi = b0_ref[0, 0]
    bo = b0_ref[0, 1]
    ga = b0_ref[0, 2]

    def step(_, xs):
        o = xs * win + bi
        o = o * wout + bo
        return xs + 0.01 * (xs * ga + o)

    out_ref[...] = lax.fori_loop(0, 100, step, x0_ref[...])


_NS = 16
_NZT = 5
_NPS = _N // _NZT
_BLK = 2000
_NBLK = _EPT // _BLK


def _sc_edge_body(row_hbm, col_hbm, nodes_hbm,
                  part_hbm,
                  svm, dvm, h0v, h1v, h2v, h3v, cv,
                  a0, a1, a2, a3, a4, rowb, colb):
    cid = lax.axis_index("c")
    sid = lax.axis_index("s")
    wid = sid * 2 + cid
    base0 = wid * _EPT
    accs = (a0, a1, a2, a3, a4)

    pltpu.sync_copy(nodes_hbm.at[pl.ds(4 * _N, _N)], svm)
    pltpu.sync_copy(nodes_hbm.at[pl.ds(5 * _N, _N)], dvm)
    pltpu.sync_copy(nodes_hbm.at[pl.ds(0 * _N, _N)], h0v)
    pltpu.sync_copy(nodes_hbm.at[pl.ds(1 * _N, _N)], h1v)
    pltpu.sync_copy(nodes_hbm.at[pl.ds(2 * _N, _N)], h2v)
    pltpu.sync_copy(nodes_hbm.at[pl.ds(3 * _N, _N)], h3v)
    pltpu.sync_copy(nodes_hbm.at[pl.ds(6 * _N, 16)], cv)

    z16 = jnp.zeros((16,), _f32)

    @pl.loop(0, _N // 16)
    def _z(i):
        o = i * 16
        a0[pl.ds(o, 16)] = z16
        a1[pl.ds(o, 16)] = z16
        a2[pl.ds(o, 16)] = z16
        a3[pl.ds(o, 16)] = z16
        a4[pl.ds(o, 16)] = z16

    cvec = cv[...]

    @pl.loop(0, _NBLK)
    def _blk(bi):
        pltpu.sync_copy(row_hbm.at[pl.ds(base0 + bi * _BLK, _BLK)], rowb)
        pltpu.sync_copy(col_hbm.at[pl.ds(base0 + bi * _BLK, _BLK)], colb)

        @pl.loop(0, _BLK // 64)
        def _grp(i):
            for u in range(4):
                o = i * 64 + u * 16
                r = rowb[pl.ds(o, 16)]
                ci = colb[pl.ds(o, 16)]
                sv = plsc.load_gather(svm, [r])
                dv = plsc.load_gather(dvm, [ci])
                al = sv + dv
                al = jnp.where(al >= 0, al, 0.2 * al)
                ex = jnp.exp(al - cvec)
                plsc.addupdate_scatter(a0, [ci],
                                       ex * plsc.load_gather(h0v, [r]))
                plsc.addupdate_scatter(a1, [ci],
                                       ex * plsc.load_gather(h1v, [r]))
                plsc.addupdate_scatter(a2, [ci],
                                       ex * plsc.load_gather(h2v, [r]))
                plsc.addupdate_scatter(a3, [ci],
                                       ex * plsc.load_gather(h3v, [r]))
                plsc.addupdate_scatter(a4, [ci], ex)

    for j in range(5):
        pltpu.sync_copy(accs[j], part_hbm.at[pl.ds((wid * 5 + j) * _N, _N)])


def _sc_gather_body(row_hbm, col_hbm, nodes_hbm, out_hbm,
                    nodev, rowb, colb, outb):
    cid = lax.axis_index("c")
    sid = lax.axis_index("s")
    wid = sid * 2 + cid
    base = wid * _EPT
    pltpu.sync_copy(row_hbm.at[pl.ds(base, _EPT)], rowb)
    pltpu.sync_copy(col_hbm.at[pl.ds(base, _EPT)], colb)
    for c in range(8):
        pltpu.sync_copy(nodes_hbm.at[pl.ds((c % 4) * _N, _N)], nodev)
        idxb = rowb if c < 4 else colb

        @pl.loop(0, _EPT // 16)
        def _grp(i):
            o = i * 16
            outb[pl.ds(o, 16)] = plsc.load_gather(nodev, [idxb[pl.ds(o, 16)]])

        pltpu.sync_copy(outb, out_hbm.at[pl.ds(c * _E + base, _EPT)])


_f32 = jnp.float32


_SC_PARAMS = pltpu.CompilerParams(needs_layout_passes=False)


def _make_sc_edge(n, e):
    mesh = plsc.VectorSubcoreMesh(core_axis_name="c", subcore_axis_name="s")
    return pl.kernel(
        _sc_edge_body,
        out_type=jax.ShapeDtypeStruct((_NW * 5 * n,), _f32),
        mesh=mesh,
        compiler_params=_SC_PARAMS,
        scratch_types=[
            pltpu.VMEM((n,), _f32), pltpu.VMEM((n,), _f32),
            pltpu.VMEM((n,), _f32), pltpu.VMEM((n,), _f32),
            pltpu.VMEM((n,), _f32), pltpu.VMEM((n,), _f32),
            pltpu.VMEM((16,), _f32),
            pltpu.VMEM((n,), _f32), pltpu.VMEM((n,), _f32),
            pltpu.VMEM((n,), _f32), pltpu.VMEM((n,), _f32),
            pltpu.VMEM((n,), _f32),
            pltpu.VMEM((_BLK,), jnp.int32), pltpu.VMEM((_BLK,), jnp.int32),
        ],
    )


def _make_sc_gather(n, e):
    mesh = plsc.VectorSubcoreMesh(core_axis_name="c", subcore_axis_name="s")
    return pl.kernel(
        _sc_gather_body,
        out_type=jax.ShapeDtypeStruct((8 * e,), _f32),
        mesh=mesh,
        compiler_params=_SC_PARAMS,
        scratch_types=[
            pltpu.VMEM((n,), _f32),
            pltpu.VMEM((_EPT,), jnp.int32), pltpu.VMEM((_EPT,), jnp.int32),
            pltpu.VMEM((_EPT,), _f32),
        ],
    )


def kernel(x, edge_index, edge_attr, batch, W1, a1s, a1d, b1, W2, a2s, a2d,
           b2, W3, a3s, a3d, b3, ew1, eb1, ew2, eb2, ew3, eb3, ew4, eb4,
           bw1, bb1, bw2, bb2, bw3, bb3, bw4, bb4):
    n, df = x.shape
    e = edge_index.shape[1]
    half = e // 2

    xT = x.T
    row = edge_index[0]
    col = edge_index[1]
    attrT = edge_attr.reshape(1, e)

    node_outs = [jax.ShapeDtypeStruct((n,), _f32)] * 6 + [
        jax.ShapeDtypeStruct((16,), _f32)]

    prep1 = pl.pallas_call(_tc_prep1_body, out_shape=node_outs)
    h0, h1, h2, h3, s, d, cvec = prep1(xT, W1, jnp.stack([a1s, a1d]))
    nodes = jnp.concatenate([h0, h1, h2, h3, s, d, cvec])

    sc_edge = _make_sc_edge(n, e)
    part = sc_edge(row, col, nodes).reshape(_NW, 5, n)

    combine = pl.pallas_call(_tc_combine_body, out_shape=node_outs)
    h0, h1, h2, h3, s, d, cvec = combine(part, b1.reshape(4, 1), W2,
                                         jnp.stack([a2s, a2d]))
    nodes = jnp.concatenate([h0, h1, h2, h3, s, d, cvec])
    part = sc_edge(row, col, nodes).reshape(_NW, 5, n)

    h0, h1, h2, h3, s, d, cvec = combine(part, b2.reshape(4, 1), W3,
                                         jnp.stack([a3s, a3d]))
    nodes = jnp.concatenate([h0, h1, h2, h3, s, d, cvec])
    part = sc_edge(row, col, nodes).reshape(_NW, 5, n)

    final_nodes = pl.pallas_call(
        _tc_final_nodes_body,
        out_shape=[jax.ShapeDtypeStruct((n,), _f32)] * 4 + [
            jax.ShapeDtypeStruct((3, 1), _f32)],
    )
    h0, h1, h2, h3, b0 = final_nodes(
        part, b3.reshape(4, 1),
        bw1, bb1.reshape(10, 1), bw2, bb2.reshape(10, 1),
        bw3, bb3.reshape(10, 1), bw4, bb4.reshape(3, 1))
    nodes = jnp.concatenate([h0, h1, h2, h3])

    sc_gather = _make_sc_gather(n, e)
    rc = sc_gather(row, col, nodes).reshape(8, e)

    eb = 16000
    grid = e // eb
    edge_mlp = pl.pallas_call(
        _tc_edge_mlp_body,
        grid=(grid,),
        in_specs=[
            pl.BlockSpec((8, eb), lambda i: (0, i)),
            pl.BlockSpec((1, eb), lambda i: (0, i)),
            pl.BlockSpec((9, 10), lambda i: (0, 0)),
            pl.BlockSpec((10, 1), lambda i: (0, 0)),
            pl.BlockSpec((10, 10), lambda i: (0, 0)),
            pl.BlockSpec((10, 1), lambda i: (0, 0)),
            pl.BlockSpec((10, 10), lambda i: (0, 0)),
            pl.BlockSpec((10, 1), lambda i: (0, 0)),
            pl.BlockSpec((10, 1), lambda i: (0, 0)),
            pl.BlockSpec((1, 1), lambda i: (0, 0)),
        ],
        out_specs=pl.BlockSpec((1, eb), lambda i: (0, i)),
        out_shape=jax.ShapeDtypeStruct((1, e), _f32),
    )
    flat2d = edge_mlp(rc, attrT, ew1, eb1.reshape(10, 1), ew2,
                      eb2.reshape(10, 1), ew3, eb3.reshape(10, 1), ew4,
                      eb4.reshape(1, 1))
    flat = flat2d.reshape(e)

    ode = pl.pallas_call(
        _tc_ode_body,
        in_specs=[
            pl.BlockSpec(memory_space=pltpu.VMEM),
            pl.BlockSpec(memory_space=pltpu.VMEM),
            pl.BlockSpec(memory_space=pltpu.SMEM),
            pl.BlockSpec(memory_space=pltpu.VMEM),
        ],
        out_shape=jax.ShapeDtypeStruct((1, df), _f32),
    )
    win = flat[0:df].reshape(1, df)
    wout = flat[half:half + df].reshape(1, df)
    xh = ode(win, wout, b0.reshape(1, 3), x[0].reshape(1, df))

    return jnp.concatenate([flat, xh.reshape(df)])


# speedup vs baseline: 1.0360x; 1.0360x over previous
import functools

import jax
import jax.numpy as jnp
from jax import lax
from jax.experimental import pallas as pl
from jax.experimental.pallas import tpu as pltpu
from jax.experimental.pallas import tpu_sc as plsc

_N = 10000
_E = 320000
_NW = 32
_EPT = _E // _NW
_CH = 80
_NCH = _EPT // _CH


def _leaky(x, s):
    return jnp.where(x >= 0, x, s * x)


def _emit_nodes(hT, h_refs, s_ref, d_ref, c_ref, a):
    sd = lax.dot_general(a, hT, (((1,), (0,)), ((), ())),
                         preferred_element_type=jnp.float32)
    for i in range(4):
        h_refs[i][...] = hT[i]
    s_ref[...] = sd[0]
    d_ref[...] = sd[1]
    m = jnp.max(sd[0:1]) + jnp.max(sd[1:2])
    c_ref[...] = jnp.full((16,), _leaky(m, 0.2), jnp.float32)


def _tc_prep1_body(xT_ref, W1_ref, a_ref, h0_ref, h1_ref, h2_ref, h3_ref,
                   s_ref, d_ref, c_ref):
    hT = lax.dot_general(W1_ref[...], xT_ref[...], (((0,), (0,)), ((), ())),
                         preferred_element_type=jnp.float32)
    _emit_nodes(hT, (h0_ref, h1_ref, h2_ref, h3_ref), s_ref, d_ref, c_ref,
                a_ref[...])


def _tc_combine_body(part_ref, b_ref, W_ref, a_ref, h0_ref, h1_ref, h2_ref,
                     h3_ref, s_ref, d_ref, c_ref):
    acc = jnp.sum(part_ref[...], axis=0)
    outT = acc[0:4] / (acc[4:5] + 1e-16) + b_ref[...]
    hT = lax.dot_general(W_ref[...], outT, (((0,), (0,)), ((), ())),
                         preferred_element_type=jnp.float32)
    _emit_nodes(hT, (h0_ref, h1_ref, h2_ref, h3_ref), s_ref, d_ref, c_ref,
                a_ref[...])


def _tc_final_nodes_body(part_ref, b_ref, bw1_ref, bb1_ref, bw2_ref, bb2_ref,
                         bw3_ref, bb3_ref, bw4_ref, bb4_ref,
                         h0_ref, h1_ref, h2_ref, h3_ref, b0_ref):
    acc = jnp.sum(part_ref[...], axis=0)
    outT = acc[0:4] / (acc[4:5] + 1e-16) + b_ref[...]
    h0_ref[...] = outT[0]
    h1_ref[...] = outT[1]
    h2_ref[...] = outT[2]
    h3_ref[...] = outT[3]
    h0 = outT[:, 0:1]
    t = _leaky(lax.dot_general(bw1_ref[...], h0, (((0,), (0,)), ((), ())),
                               preferred_element_type=jnp.float32)
               + bb1_ref[...], 0.01)
    t = _leaky(lax.dot_general(bw2_ref[...], t, (((0,), (0,)), ((), ())),
                               preferred_element_type=jnp.float32)
               + bb2_ref[...], 0.01)
    t = _leaky(lax.dot_general(bw3_ref[...], t, (((0,), (0,)), ((), ())),
                               preferred_element_type=jnp.float32)
               + bb3_ref[...], 0.01)
    b0 = (lax.dot_general(bw4_ref[...], t, (((0,), (0,)), ((), ())),
                          preferred_element_type=jnp.float32)
          + bb4_ref[...])
    b0_ref[...] = b0


def _tc_edge_mlp_body(rc_ref, at_ref, ew1_ref, eb1_ref, ew2_ref, eb2_ref,
                      ew3_ref, eb3_ref, ew4_ref, eb4_ref, out_ref):
    ew1 = ew1_ref[...]
    t = lax.dot_general(ew1[0:8], rc_ref[...], (((0,), (0,)), ((), ())),
                        preferred_element_type=jnp.float32)
    t = t + lax.dot_general(ew1[8:9], at_ref[...], (((0,), (0,)), ((), ())),
                            preferred_element_type=jnp.float32)
    t = _leaky(t + eb1_ref[...], 0.01)
    t = _leaky(lax.dot_general(ew2_ref[...], t, (((0,), (0,)), ((), ())),
                               preferred_element_type=jnp.float32)
               + eb2_ref[...], 0.01)
    t = _leaky(lax.dot_general(ew3_ref[...], t, (((0,), (0,)), ((), ())),
                               preferred_element_type=jnp.float32)
               + eb3_ref[...], 0.01)
    out_ref[...] = (lax.dot_general(ew4_ref[...], t, (((0,), (0,)), ((), ())),
                                    preferred_element_type=jnp.float32)
                    + eb4_ref[...])


def _tc_ode_body(win_ref, wout_ref, b0_ref, x0_ref, out_ref):
    win = win_ref[...]
    wout = wout_ref[...]
    bi = b0_ref[0, 0]
    bo = b0_ref[0, 1]
    ga = b0_ref[0, 2]

    def step(_, xs):
        o = xs * win + bi
        o = o * wout + bo
        return xs + 0.01 * (xs * ga + o)

    out_ref[...] = lax.fori_loop(0, 100, step, x0_ref[...])


_NS = 16
_NZT = 5
_NPS = _N // _NZT
_BLK = 2000
_NBLK = _EPT // _BLK


def _sc_edge_body(row_hbm, col_hbm, col2_hbm, nodes_hbm, zero_hbm,
                  part_hbm,
                  svm, dvm, h0v, h1v, h2v, h3v, cv,
                  rowb, colb, colb2, zbuf,
                  st0, st1, st2, st3, st4,
                  sem0, sem1, sem2, sem3, sem4,
                  a0, a1, a2, a3, a4):
    cid = lax.axis_index("c")
    sid = lax.axis_index("s")
    wid = sid * 2 + cid
    base0 = wid * _EPT
    accs = (a0, a1, a2, a3, a4)
    sts = (st0, st1, st2, st3, st4)
    sems = (sem0, sem1, sem2, sem3, sem4)

    pltpu.sync_copy(nodes_hbm.at[pl.ds(4 * _N, _N)], svm)
    pltpu.sync_copy(nodes_hbm.at[pl.ds(5 * _N, _N)], dvm)
    pltpu.sync_copy(nodes_hbm.at[pl.ds(0 * _N, _N)], h0v)
    pltpu.sync_copy(nodes_hbm.at[pl.ds(1 * _N, _N)], h1v)
    pltpu.sync_copy(nodes_hbm.at[pl.ds(2 * _N, _N)], h2v)
    pltpu.sync_copy(nodes_hbm.at[pl.ds(3 * _N, _N)], h3v)
    pltpu.sync_copy(nodes_hbm.at[pl.ds(6 * _N, 16)], cv)
    pltpu.sync_copy(col2_hbm.at[pl.ds(wid * 128, 128), :], colb2)

    @pl.when(sid == 0)
    def _():
        pltpu.sync_copy(zero_hbm, zbuf)
        for acc in accs:
            pltpu.sync_copy(zbuf, acc)

    plsc.subcore_barrier()

    cvec = cv[...]

    @pl.loop(0, _NBLK)
    def _blk(bi):
        pltpu.sync_copy(row_hbm.at[pl.ds(base0 + bi * _BLK, _BLK)], rowb)
        pltpu.sync_copy(col_hbm.at[pl.ds(base0 + bi * _BLK, _BLK)], colb)

        @pl.loop(0, _BLK // (_CH * 5))
        def _burst(bu):
            descs = []
            for b in range(5):
                c = bu * 5 + b
                k = bi * (_BLK // _CH) + c
                for g in range(_CH // 16):
                    o = c * _CH + g * 16
                    r = rowb[pl.ds(o, 16)]
                    ci = colb[pl.ds(o, 16)]
                    sv = plsc.load_gather(svm, [r])
                    dv = plsc.load_gather(dvm, [ci])
                    al = sv + dv
                    al = jnp.where(al >= 0, al, 0.2 * al)
                    ex = jnp.exp(al - cvec)
                    og = g * 16
                    st0[b, pl.ds(og, 16)] = ex * plsc.load_gather(h0v, [r])
                    st1[b, pl.ds(og, 16)] = ex * plsc.load_gather(h1v, [r])
                    st2[b, pl.ds(og, 16)] = ex * plsc.load_gather(h2v, [r])
                    st3[b, pl.ds(og, 16)] = ex * plsc.load_gather(h3v, [r])
                    st4[b, pl.ds(og, 16)] = ex
                for j in range(5):
                    descs.append(pltpu.async_copy(
                        sts[j].at[b], accs[j].at[colb2.at[k]], sems[b],
                        add=True))
            for dsc in descs:
                dsc.wait()

    plsc.subcore_barrier()

    @pl.when(sid == 0)
    def _():
        for j, acc in enumerate(accs):
            pltpu.sync_copy(acc, zbuf)
            pltpu.sync_copy(zbuf, part_hbm.at[pl.ds((cid * 5 + j) * _N, _N)])


def _sc_gather_body(row_hbm, col_hbm, nodes_hbm, out_hbm,
                    nodev, rowb, colb, outb):
    cid = lax.axis_index("c")
    sid = lax.axis_index("s")
    wid = sid * 2 + cid
    base = wid * _EPT
    pltpu.sync_copy(row_hbm.at[pl.ds(base, _EPT)], rowb)
    pltpu.sync_copy(col_hbm.at[pl.ds(base, _EPT)], colb)
    for c in range(8):
        pltpu.sync_copy(nodes_hbm.at[pl.ds((c % 4) * _N, _N)], nodev)
        idxb = rowb if c < 4 else colb

        @pl.loop(0, _EPT // 16)
        def _grp(i):
            o = i * 16
            outb[pl.ds(o, 16)] = plsc.load_gather(nodev, [idxb[pl.ds(o, 16)]])

        pltpu.sync_copy(outb, out_hbm.at[pl.ds(c * _E + base, _EPT)])


_f32 = jnp.float32


_SC_PARAMS = pltpu.CompilerParams(needs_layout_passes=False)


def _make_sc_edge(n, e):
    mesh = plsc.VectorSubcoreMesh(core_axis_name="c", subcore_axis_name="s")
    return pl.kernel(
        _sc_edge_body,
        out_type=jax.ShapeDtypeStruct((2 * 5 * n,), _f32),
        mesh=mesh,
        compiler_params=_SC_PARAMS,
        scratch_types=[
            pltpu.VMEM((n,), _f32), pltpu.VMEM((n,), _f32),
            pltpu.VMEM((n,), _f32), pltpu.VMEM((n,), _f32),
            pltpu.VMEM((n,), _f32), pltpu.VMEM((n,), _f32),
            pltpu.VMEM((16,), _f32),
            pltpu.VMEM((_BLK,), jnp.int32), pltpu.VMEM((_BLK,), jnp.int32),
            pltpu.VMEM((128, _CH), jnp.int32), pltpu.VMEM((n,), _f32),
            pltpu.VMEM((5, _CH), _f32), pltpu.VMEM((5, _CH), _f32),
            pltpu.VMEM((5, _CH), _f32), pltpu.VMEM((5, _CH), _f32),
            pltpu.VMEM((5, _CH), _f32),
            pltpu.SemaphoreType.DMA, pltpu.SemaphoreType.DMA,
            pltpu.SemaphoreType.DMA, pltpu.SemaphoreType.DMA,
            pltpu.SemaphoreType.DMA,
            pltpu.VMEM_SHARED((n,), _f32), pltpu.VMEM_SHARED((n,), _f32),
            pltpu.VMEM_SHARED((n,), _f32), pltpu.VMEM_SHARED((n,), _f32),
            pltpu.VMEM_SHARED((n,), _f32),
        ],
    )


def _make_sc_gather(n, e):
    mesh = plsc.VectorSubcoreMesh(core_axis_name="c", subcore_axis_name="s")
    return pl.kernel(
        _sc_gather_body,
        out_type=jax.ShapeDtypeStruct((8 * e,), _f32),
        mesh=mesh,
        compiler_params=_SC_PARAMS,
        scratch_types=[
            pltpu.VMEM((n,), _f32),
            pltpu.VMEM((_EPT,), jnp.int32), pltpu.VMEM((_EPT,), jnp.int32),
            pltpu.VMEM((_EPT,), _f32),
        ],
    )


def kernel(x, edge_index, edge_attr, batch, W1, a1s, a1d, b1, W2, a2s, a2d,
           b2, W3, a3s, a3d, b3, ew1, eb1, ew2, eb2, ew3, eb3, ew4, eb4,
           bw1, bb1, bw2, bb2, bw3, bb3, bw4, bb4):
    n, df = x.shape
    e = edge_index.shape[1]
    half = e // 2

    xT = x.T
    row = edge_index[0]
    col = edge_index[1]
    attrT = edge_attr.reshape(1, e)
    col2 = jnp.pad(col.reshape(_NW, _NCH, _CH),
                   ((0, 0), (0, 128 - _NCH), (0, 0))).reshape(_NW * 128, _CH)
    zeros_n = jnp.zeros((n,), _f32)

    node_outs = [jax.ShapeDtypeStruct((n,), _f32)] * 6 + [
        jax.ShapeDtypeStruct((16,), _f32)]

    prep1 = pl.pallas_call(_tc_prep1_body, out_shape=node_outs)
    h0, h1, h2, h3, s, d, cvec = prep1(xT, W1, jnp.stack([a1s, a1d]))
    nodes = jnp.concatenate([h0, h1, h2, h3, s, d, cvec])

    sc_edge = _make_sc_edge(n, e)
    part = sc_edge(row, col, col2, nodes, zeros_n).reshape(2, 5, n)

    combine = pl.pallas_call(_tc_combine_body, out_shape=node_outs)
    h0, h1, h2, h3, s, d, cvec = combine(part, b1.reshape(4, 1), W2,
                                         jnp.stack([a2s, a2d]))
    nodes = jnp.concatenate([h0, h1, h2, h3, s, d, cvec])
    part = sc_edge(row, col, col2, nodes, zeros_n).reshape(2, 5, n)

    h0, h1, h2, h3, s, d, cvec = combine(part, b2.reshape(4, 1), W3,
                                         jnp.stack([a3s, a3d]))
    nodes = jnp.concatenate([h0, h1, h2, h3, s, d, cvec])
    part = sc_edge(row, col, col2, nodes, zeros_n).reshape(2, 5, n)

    final_nodes = pl.pallas_call(
        _tc_final_nodes_body,
        out_shape=[jax.ShapeDtypeStruct((n,), _f32)] * 4 + [
            jax.ShapeDtypeStruct((3, 1), _f32)],
    )
    h0, h1, h2, h3, b0 = final_nodes(
        part, b3.reshape(4, 1),
        bw1, bb1.reshape(10, 1), bw2, bb2.reshape(10, 1),
        bw3, bb3.reshape(10, 1), bw4, bb4.reshape(3, 1))
    nodes = jnp.concatenate([h0, h1, h2, h3])

    sc_gather = _make_sc_gather(n, e)
    rc = sc_gather(row, col, nodes).reshape(8, e)

    eb = 16000
    grid = e // eb
    edge_mlp = pl.pallas_call(
        _tc_edge_mlp_body,
        grid=(grid,),
        in_specs=[
            pl.BlockSpec((8, eb), lambda i: (0, i)),
            pl.BlockSpec((1, eb), lambda i: (0, i)),
            pl.BlockSpec((9, 10), lambda i: (0, 0)),
            pl.BlockSpec((10, 1), lambda i: (0, 0)),
            pl.BlockSpec((10, 10), lambda i: (0, 0)),
            pl.BlockSpec((10, 1), lambda i: (0, 0)),
            pl.BlockSpec((10, 10), lambda i: (0, 0)),
            pl.BlockSpec((10, 1), lambda i: (0, 0)),
            pl.BlockSpec((10, 1), lambda i: (0, 0)),
            pl.BlockSpec((1, 1), lambda i: (0, 0)),
        ],
        out_specs=pl.BlockSpec((1, eb), lambda i: (0, i)),
        out_shape=jax.ShapeDtypeStruct((1, e), _f32),
    )
    flat2d = edge_mlp(rc, attrT, ew1, eb1.reshape(10, 1), ew2,
                      eb2.reshape(10, 1), ew3, eb3.reshape(10, 1), ew4,
                      eb4.reshape(1, 1))
    flat = flat2d.reshape(e)

    ode = pl.pallas_call(
        _tc_ode_body,
        in_specs=[
            pl.BlockSpec(memory_space=pltpu.VMEM),
            pl.BlockSpec(memory_space=pltpu.VMEM),
            pl.BlockSpec(memory_space=pltpu.SMEM),
            pl.BlockSpec(memory_space=pltpu.VMEM),
        ],
        out_shape=jax.ShapeDtypeStruct((1, df), _f32),
    )
    win = flat[0:df].reshape(1, df)
    wout = flat[half:half + df].reshape(1, df)
    xh = ode(win, wout, b0.reshape(1, 3), x[0].reshape(1, df))

    return jnp.concatenate([flat, xh.reshape(df)])


# speedup vs baseline: 1.0854x; 1.0476x over previous
import functools

import jax
import jax.numpy as jnp
from jax import lax
from jax.experimental import pallas as pl
from jax.experimental.pallas import tpu as pltpu
from jax.experimental.pallas import tpu_sc as plsc

_N = 10000
_E = 320000
_NW = 32
_EPT = _E // _NW
_CH = 80
_NCH = _EPT // _CH


def _leaky(x, s):
    return jnp.where(x >= 0, x, s * x)


def _emit_nodes(hT, h_refs, s_ref, d_ref, c_ref, a):
    sd = lax.dot_general(a, hT, (((1,), (0,)), ((), ())),
                         preferred_element_type=jnp.float32)
    for i in range(4):
        h_refs[i][...] = hT[i]
    s_ref[...] = sd[0]
    d_ref[...] = sd[1]
    m = jnp.max(sd[0:1]) + jnp.max(sd[1:2])
    c_ref[...] = jnp.full((16,), _leaky(m, 0.2), jnp.float32)


def _tc_prep1_body(xT_ref, W1_ref, a_ref, h0_ref, h1_ref, h2_ref, h3_ref,
                   s_ref, d_ref, c_ref):
    hT = lax.dot_general(W1_ref[...], xT_ref[...], (((0,), (0,)), ((), ())),
                         preferred_element_type=jnp.float32)
    _emit_nodes(hT, (h0_ref, h1_ref, h2_ref, h3_ref), s_ref, d_ref, c_ref,
                a_ref[...])


def _tc_combine_body(part_ref, b_ref, W_ref, a_ref, h0_ref, h1_ref, h2_ref,
                     h3_ref, s_ref, d_ref, c_ref):
    acc = jnp.sum(part_ref[...], axis=0)
    outT = acc[0:4] / (acc[4:5] + 1e-16) + b_ref[...]
    hT = lax.dot_general(W_ref[...], outT, (((0,), (0,)), ((), ())),
                         preferred_element_type=jnp.float32)
    _emit_nodes(hT, (h0_ref, h1_ref, h2_ref, h3_ref), s_ref, d_ref, c_ref,
                a_ref[...])


def _tc_final_nodes_body(part_ref, b_ref, bw1_ref, bb1_ref, bw2_ref, bb2_ref,
                         bw3_ref, bb3_ref, bw4_ref, bb4_ref,
                         h0_ref, h1_ref, h2_ref, h3_ref, b0_ref):
    acc = jnp.sum(part_ref[...], axis=0)
    outT = acc[0:4] / (acc[4:5] + 1e-16) + b_ref[...]
    h0_ref[...] = outT[0]
    h1_ref[...] = outT[1]
    h2_ref[...] = outT[2]
    h3_ref[...] = outT[3]
    h0 = outT[:, 0:1]
    t = _leaky(lax.dot_general(bw1_ref[...], h0, (((0,), (0,)), ((), ())),
                               preferred_element_type=jnp.float32)
               + bb1_ref[...], 0.01)
    t = _leaky(lax.dot_general(bw2_ref[...], t, (((0,), (0,)), ((), ())),
                               preferred_element_type=jnp.float32)
               + bb2_ref[...], 0.01)
    t = _leaky(lax.dot_general(bw3_ref[...], t, (((0,), (0,)), ((), ())),
                               preferred_element_type=jnp.float32)
               + bb3_ref[...], 0.01)
    b0 = (lax.dot_general(bw4_ref[...], t, (((0,), (0,)), ((), ())),
                          preferred_element_type=jnp.float32)
          + bb4_ref[...])
    b0_ref[...] = b0


def _tc_edge_mlp_body(rc_ref, at_ref, ew1_ref, eb1_ref, ew2_ref, eb2_ref,
                      ew3_ref, eb3_ref, ew4_ref, eb4_ref, out_ref):
    ew1 = ew1_ref[...]
    t = lax.dot_general(ew1[0:8], rc_ref[...], (((0,), (0,)), ((), ())),
                        preferred_element_type=jnp.float32)
    t = t + lax.dot_general(ew1[8:9], at_ref[...], (((0,), (0,)), ((), ())),
                            preferred_element_type=jnp.float32)
    t = _leaky(t + eb1_ref[...], 0.01)
    t = _leaky(lax.dot_general(ew2_ref[...], t, (((0,), (0,)), ((), ())),
                               preferred_element_type=jnp.float32)
               + eb2_ref[...], 0.01)
    t = _leaky(lax.dot_general(ew3_ref[...], t, (((0,), (0,)), ((), ())),
                               preferred_element_type=jnp.float32)
               + eb3_ref[...], 0.01)
    out_ref[...] = (lax.dot_general(ew4_ref[...], t, (((0,), (0,)), ((), ())),
                                    preferred_element_type=jnp.float32)
                    + eb4_ref[...])


def _tc_ode_body(win_ref, wout_ref, b0_ref, x0_ref, out_ref):
    win = win_ref[...]
    wout = wout_ref[...]
    bi = b0_ref[0, 0]
    bo = b0_ref[0, 1]
    ga = b0_ref[0, 2]

    def step(_, xs):
        o = xs * win + bi
        o = o * wout + bo
        return xs + 0.01 * (xs * ga + o)

    out_ref[...] = lax.fori_loop(0, 100, step, x0_ref[...])


_NS = 16
_NZT = 5
_NPS = _N // _NZT
_BLK = 2000
_NBLK = _EPT // _BLK


def _sc_edge_body(row_hbm, col_hbm, col2_hbm, nodes_hbm, zero_hbm,
                  part_hbm,
                  svm, dvm, h0v, h1v, h2v, h3v, cv,
                  rowb, colb, colb2, zbuf,
                  st0, st1, st2, st3, st4,
                  sem0, sem1, sem2, sem3, sem4,
                  a0, a1, a2, a3, a4):
    cid = lax.axis_index("c")
    sid = lax.axis_index("s")
    wid = sid * 2 + cid
    base0 = wid * _EPT
    accs = (a0, a1, a2, a3, a4)
    sts = (st0, st1, st2, st3, st4)
    sems = (sem0, sem1, sem2, sem3, sem4)

    pltpu.sync_copy(nodes_hbm.at[pl.ds(4 * _N, _N)], svm)
    pltpu.sync_copy(nodes_hbm.at[pl.ds(5 * _N, _N)], dvm)
    pltpu.sync_copy(nodes_hbm.at[pl.ds(0 * _N, _N)], h0v)
    pltpu.sync_copy(nodes_hbm.at[pl.ds(1 * _N, _N)], h1v)
    pltpu.sync_copy(nodes_hbm.at[pl.ds(2 * _N, _N)], h2v)
    pltpu.sync_copy(nodes_hbm.at[pl.ds(3 * _N, _N)], h3v)
    pltpu.sync_copy(nodes_hbm.at[pl.ds(6 * _N, 16)], cv)
    pltpu.sync_copy(col2_hbm.at[pl.ds(wid * 128, 128), :], colb2)

    @pl.when(sid == 0)
    def _():
        pltpu.sync_copy(zero_hbm, zbuf)
        for acc in accs:
            pltpu.sync_copy(zbuf, acc)

    plsc.subcore_barrier()

    cvec = cv[...]

    @pl.loop(0, _NBLK)
    def _blk(bi):
        pltpu.sync_copy(row_hbm.at[pl.ds(base0 + bi * _BLK, _BLK)], rowb)
        pltpu.sync_copy(col_hbm.at[pl.ds(base0 + bi * _BLK, _BLK)], colb)

        @pl.loop(0, _BLK // (_CH * 5))
        def _burst(bu):
            descs = []
            for b in range(5):
                c = bu * 5 + b
                k = bi * (_BLK // _CH) + c
                for g in range(_CH // 16):
                    o = c * _CH + g * 16
                    r = rowb[pl.ds(o, 16)]
                    ci = colb[pl.ds(o, 16)]
                    sv = plsc.load_gather(svm, [r])
                    dv = plsc.load_gather(dvm, [ci])
                    al = sv + dv
                    al = jnp.where(al >= 0, al, 0.2 * al)
                    ex = jnp.exp(al - cvec)
                    og = g * 16
                    st0[b, pl.ds(og, 16)] = ex * plsc.load_gather(h0v, [r])
                    st1[b, pl.ds(og, 16)] = ex * plsc.load_gather(h1v, [r])
                    st2[b, pl.ds(og, 16)] = ex * plsc.load_gather(h2v, [r])
                    st3[b, pl.ds(og, 16)] = ex * plsc.load_gather(h3v, [r])
                    st4[b, pl.ds(og, 16)] = ex
                for j in range(5):
                    descs.append(pltpu.async_copy(
                        sts[j].at[b], accs[j].at[colb2.at[k]], sems[b],
                        add=True))
            for dsc in descs:
                dsc.wait()

    plsc.subcore_barrier()

    @pl.when(sid == 0)
    def _():
        for j, acc in enumerate(accs):
            pltpu.sync_copy(acc, zbuf)
            pltpu.sync_copy(zbuf, part_hbm.at[pl.ds((cid * 5 + j) * _N, _N)])


def _sc_gather_body(row_hbm, col_hbm, nodes_hbm, out_hbm,
                    n0, n1, n2, n3, idxb, ob0, ob1, ob2, ob3):
    cid = lax.axis_index("c")
    sid = lax.axis_index("s")
    wid = sid * 2 + cid
    base = wid * _EPT
    planes = (n0, n1, n2, n3)
    obs = (ob0, ob1, ob2, ob3)
    for k in range(4):
        pltpu.sync_copy(nodes_hbm.at[pl.ds(k * _N, _N)], planes[k])
    for p, src in enumerate((row_hbm, col_hbm)):
        pltpu.sync_copy(src.at[pl.ds(base, _EPT)], idxb)

        @pl.loop(0, _EPT // 16)
        def _grp(i):
            o = i * 16
            r = idxb[pl.ds(o, 16)]
            ob0[pl.ds(o, 16)] = plsc.load_gather(n0, [r])
            ob1[pl.ds(o, 16)] = plsc.load_gather(n1, [r])
            ob2[pl.ds(o, 16)] = plsc.load_gather(n2, [r])
            ob3[pl.ds(o, 16)] = plsc.load_gather(n3, [r])

        for k in range(4):
            pltpu.sync_copy(
                obs[k], out_hbm.at[pl.ds((p * 4 + k) * _E + base, _EPT)])


_f32 = jnp.float32


_SC_PARAMS = pltpu.CompilerParams(needs_layout_passes=False)


def _make_sc_edge(n, e):
    mesh = plsc.VectorSubcoreMesh(core_axis_name="c", subcore_axis_name="s")
    return pl.kernel(
        _sc_edge_body,
        out_type=jax.ShapeDtypeStruct((2 * 5 * n,), _f32),
        mesh=mesh,
        compiler_params=_SC_PARAMS,
        scratch_types=[
            pltpu.VMEM((n,), _f32), pltpu.VMEM((n,), _f32),
            pltpu.VMEM((n,), _f32), pltpu.VMEM((n,), _f32),
            pltpu.VMEM((n,), _f32), pltpu.VMEM((n,), _f32),
            pltpu.VMEM((16,), _f32),
            pltpu.VMEM((_BLK,), jnp.int32), pltpu.VMEM((_BLK,), jnp.int32),
            pltpu.VMEM((128, _CH), jnp.int32), pltpu.VMEM((n,), _f32),
            pltpu.VMEM((5, _CH), _f32), pltpu.VMEM((5, _CH), _f32),
            pltpu.VMEM((5, _CH), _f32), pltpu.VMEM((5, _CH), _f32),
            pltpu.VMEM((5, _CH), _f32),
            pltpu.SemaphoreType.DMA, pltpu.SemaphoreType.DMA,
            pltpu.SemaphoreType.DMA, pltpu.SemaphoreType.DMA,
            pltpu.SemaphoreType.DMA,
            pltpu.VMEM_SHARED((n,), _f32), pltpu.VMEM_SHARED((n,), _f32),
            pltpu.VMEM_SHARED((n,), _f32), pltpu.VMEM_SHARED((n,), _f32),
            pltpu.VMEM_SHARED((n,), _f32),
        ],
    )


def _make_sc_gather(n, e):
    mesh = plsc.VectorSubcoreMesh(core_axis_name="c", subcore_axis_name="s")
    return pl.kernel(
        _sc_gather_body,
        out_type=jax.ShapeDtypeStruct((8 * e,), _f32),
        mesh=mesh,
        compiler_params=_SC_PARAMS,
        scratch_types=[
            pltpu.VMEM((n,), _f32), pltpu.VMEM((n,), _f32),
            pltpu.VMEM((n,), _f32), pltpu.VMEM((n,), _f32),
            pltpu.VMEM((_EPT,), jnp.int32),
            pltpu.VMEM((_EPT,), _f32), pltpu.VMEM((_EPT,), _f32),
            pltpu.VMEM((_EPT,), _f32), pltpu.VMEM((_EPT,), _f32),
        ],
    )


def kernel(x, edge_index, edge_attr, batch, W1, a1s, a1d, b1, W2, a2s, a2d,
           b2, W3, a3s, a3d, b3, ew1, eb1, ew2, eb2, ew3, eb3, ew4, eb4,
           bw1, bb1, bw2, bb2, bw3, bb3, bw4, bb4):
    n, df = x.shape
    e = edge_index.shape[1]
    half = e // 2

    xT = x.T
    row = edge_index[0]
    col = edge_index[1]
    attrT = edge_attr.reshape(1, e)
    col2 = jnp.pad(col.reshape(_NW, _NCH, _CH),
                   ((0, 0), (0, 128 - _NCH), (0, 0))).reshape(_NW * 128, _CH)
    zeros_n = jnp.zeros((n,), _f32)

    node_outs = [jax.ShapeDtypeStruct((n,), _f32)] * 6 + [
        jax.ShapeDtypeStruct((16,), _f32)]

    prep1 = pl.pallas_call(_tc_prep1_body, out_shape=node_outs)
    h0, h1, h2, h3, s, d, cvec = prep1(xT, W1, jnp.stack([a1s, a1d]))
    nodes = jnp.concatenate([h0, h1, h2, h3, s, d, cvec])

    sc_edge = _make_sc_edge(n, e)
    part = sc_edge(row, col, col2, nodes, zeros_n).reshape(2, 5, n)

    combine = pl.pallas_call(_tc_combine_body, out_shape=node_outs)
    h0, h1, h2, h3, s, d, cvec = combine(part, b1.reshape(4, 1), W2,
                                         jnp.stack([a2s, a2d]))
    nodes = jnp.concatenate([h0, h1, h2, h3, s, d, cvec])
    part = sc_edge(row, col, col2, nodes, zeros_n).reshape(2, 5, n)

    h0, h1, h2, h3, s, d, cvec = combine(part, b2.reshape(4, 1), W3,
                                         jnp.stack([a3s, a3d]))
    nodes = jnp.concatenate([h0, h1, h2, h3, s, d, cvec])
    part = sc_edge(row, col, col2, nodes, zeros_n).reshape(2, 5, n)

    final_nodes = pl.pallas_call(
        _tc_final_nodes_body,
        out_shape=[jax.ShapeDtypeStruct((n,), _f32)] * 4 + [
            jax.ShapeDtypeStruct((3, 1), _f32)],
    )
    h0, h1, h2, h3, b0 = final_nodes(
        part, b3.reshape(4, 1),
        bw1, bb1.reshape(10, 1), bw2, bb2.reshape(10, 1),
        bw3, bb3.reshape(10, 1), bw4, bb4.reshape(3, 1))
    nodes = jnp.concatenate([h0, h1, h2, h3])

    sc_gather = _make_sc_gather(n, e)
    rc = sc_gather(row, col, nodes).reshape(8, e)

    eb = 16000
    grid = e // eb
    edge_mlp = pl.pallas_call(
        _tc_edge_mlp_body,
        grid=(grid,),
        in_specs=[
            pl.BlockSpec((8, eb), lambda i: (0, i)),
            pl.BlockSpec((1, eb), lambda i: (0, i)),
            pl.BlockSpec((9, 10), lambda i: (0, 0)),
            pl.BlockSpec((10, 1), lambda i: (0, 0)),
            pl.BlockSpec((10, 10), lambda i: (0, 0)),
            pl.BlockSpec((10, 1), lambda i: (0, 0)),
            pl.BlockSpec((10, 10), lambda i: (0, 0)),
            pl.BlockSpec((10, 1), lambda i: (0, 0)),
            pl.BlockSpec((10, 1), lambda i: (0, 0)),
            pl.BlockSpec((1, 1), lambda i: (0, 0)),
        ],
        out_specs=pl.BlockSpec((1, eb), lambda i: (0, i)),
        out_shape=jax.ShapeDtypeStruct((1, e), _f32),
    )
    flat2d = edge_mlp(rc, attrT, ew1, eb1.reshape(10, 1), ew2,
                      eb2.reshape(10, 1), ew3, eb3.reshape(10, 1), ew4,
                      eb4.reshape(1, 1))
    flat = flat2d.reshape(e)

    ode = pl.pallas_call(
        _tc_ode_body,
        in_specs=[
            pl.BlockSpec(memory_space=pltpu.VMEM),
            pl.BlockSpec(memory_space=pltpu.VMEM),
            pl.BlockSpec(memory_space=pltpu.SMEM),
            pl.BlockSpec(memory_space=pltpu.VMEM),
        ],
        out_shape=jax.ShapeDtypeStruct((1, df), _f32),
    )
    win = flat[0:df].reshape(1, df)
    wout = flat[half:half + df].reshape(1, df)
    xh = ode(win, wout, b0.reshape(1, 3), x[0].reshape(1, df))

    return jnp.concatenate([flat, xh.reshape(df)])
